# baseline (speedup 1.0000x reference)
import jax
import jax.numpy as jnp
from jax import lax
from jax.experimental import pallas as pl
from jax.experimental.pallas import tpu as pltpu
from jax.experimental.pallas import tpu_sc as plsc

EMB = 64
LANES = 16


def _sc_lookup(idx_hbm, table_hbm, out_hbm, idx_v, row_v, out_v, sem):
    wid = lax.axis_index("s") * 2 + lax.axis_index("c")

    @pl.when(wid == 0)
    def _():
        pltpu.sync_copy(idx_hbm, idx_v)
        iv = idx_v[:]
        pair, half = iv[0], iv[1]
        base = pl.multiple_of(pair * (2 * EMB), 2 * EMB)
        pltpu.sync_copy(table_hbm.at[pl.ds(base, 2 * EMB)], row_v)
        off = half * EMB
        for k in range(EMB // LANES):
            chunk = row_v[pl.ds(off + k * LANES, LANES)]
            out_v[pl.ds(k * LANES, LANES)] = chunk
            out_v[pl.ds(EMB + k * LANES, LANES)] = chunk
        pltpu.sync_copy(out_v, out_hbm)


def kernel(table, token):
    rows, emb = table.shape
    flat = table.reshape(rows * emb)
    tok = jnp.asarray(token, jnp.int32)
    idx = jnp.concatenate([
        (tok // 2).reshape(1),
        (tok % 2).reshape(1),
        jnp.zeros((LANES - 2,), jnp.int32),
    ])
    out = pl.kernel(
        _sc_lookup,
        out_type=jax.ShapeDtypeStruct((2 * EMB,), jnp.float32),
        mesh=plsc.VectorSubcoreMesh(core_axis_name="c", subcore_axis_name="s"),
        scratch_types=[
            pltpu.VMEM((LANES,), jnp.int32),
            pltpu.VMEM((2 * EMB,), jnp.float32),
            pltpu.VMEM((2 * EMB,), jnp.float32),
            pltpu.SemaphoreType.DMA,
        ],
    )(idx, flat)
    return out[:EMB]


# speedup vs baseline: 29.8731x; 29.8731x over previous
import jax
import jax.numpy as jnp
from jax import lax
from jax.experimental import pallas as pl
from jax.experimental.pallas import tpu as pltpu
from jax.experimental.pallas import tpu_sc as plsc

EMB = 64
LANES = 16
BLK = 128


def _sc_lookup(blk_hbm, lane_hbm, table_hbm, out_hbm, blk_v, lane_v, tile_v,
               out_v, sem):
    wid = lax.axis_index("s") * 2 + lax.axis_index("c")

    @pl.when(wid == 0)
    def _():
        pltpu.sync_copy(blk_hbm, blk_v)
        pltpu.sync_copy(lane_hbm, lane_v)
        blk = blk_v[:][0]
        base = pl.multiple_of(blk * BLK, BLK)
        pltpu.sync_copy(table_hbm.at[:, pl.ds(base, BLK)], tile_v)
        lane = lane_v[:]
        for k in range(EMB // LANES):
            rows = lax.iota(jnp.int32, LANES) + k * LANES
            chunk = plsc.load_gather(tile_v, [rows, lane])
            out_v[pl.ds(k * LANES, LANES)] = chunk
            out_v[pl.ds(EMB + k * LANES, LANES)] = chunk
        pltpu.sync_copy(out_v, out_hbm)


def kernel(table, token):
    table_t = table.T
    tok = jnp.asarray(token, jnp.int32)
    blk = jnp.broadcast_to((tok // BLK).reshape(1), (LANES,))
    lane = jnp.broadcast_to((tok % BLK).reshape(1), (LANES,))
    out = pl.kernel(
        _sc_lookup,
        out_type=jax.ShapeDtypeStruct((2 * EMB,), jnp.float32),
        mesh=plsc.VectorSubcoreMesh(core_axis_name="c", subcore_axis_name="s"),
        scratch_types=[
            pltpu.VMEM((LANES,), jnp.int32),
            pltpu.VMEM((LANES,), jnp.int32),
            pltpu.VMEM((EMB, BLK), jnp.float32),
            pltpu.VMEM((2 * EMB,), jnp.float32),
            pltpu.SemaphoreType.DMA,
        ],
        compiler_params=pltpu.CompilerParams(needs_layout_passes=False),
    )(blk, lane, table_t)
    return out[:EMB]


# speedup vs baseline: 30.7107x; 1.0280x over previous
import jax
import jax.numpy as jnp
from jax import lax
from jax.experimental import pallas as pl
from jax.experimental.pallas import tpu as pltpu
from jax.experimental.pallas import tpu_sc as plsc

EMB = 64
LANES = 16
BLK = 128


def _sc_lookup(tok_hbm, table_hbm, out_hbm, tok_v, tile_v, out_v, sem):
    wid = lax.axis_index("s") * 2 + lax.axis_index("c")

    @pl.when(wid == 0)
    def _():
        pltpu.sync_copy(tok_hbm, tok_v)
        tv = tok_v[:]
        lane = lax.rem(tv, jnp.full((LANES,), BLK, jnp.int32))
        blk = lax.div(tv, jnp.full((LANES,), BLK, jnp.int32))[0]
        base = pl.multiple_of(blk * BLK, BLK)
        pltpu.sync_copy(table_hbm.at[:, pl.ds(base, BLK)], tile_v)
        for k in range(EMB // LANES):
            rows = lax.iota(jnp.int32, LANES) + k * LANES
            chunk = plsc.load_gather(tile_v, [rows, lane])
            out_v[pl.ds(k * LANES, LANES)] = chunk
            out_v[pl.ds(EMB + k * LANES, LANES)] = chunk
        pltpu.sync_copy(out_v, out_hbm)


def kernel(table, token):
    table_t = table.T
    tok16 = jnp.broadcast_to(jnp.asarray(token, jnp.int32).reshape(1), (LANES,))
    out = pl.kernel(
        _sc_lookup,
        out_type=jax.ShapeDtypeStruct((2 * EMB,), jnp.float32),
        mesh=plsc.VectorSubcoreMesh(core_axis_name="c", subcore_axis_name="s"),
        scratch_types=[
            pltpu.VMEM((LANES,), jnp.int32),
            pltpu.VMEM((EMB, BLK), jnp.float32),
            pltpu.VMEM((2 * EMB,), jnp.float32),
            pltpu.SemaphoreType.DMA,
        ],
        compiler_params=pltpu.CompilerParams(
            needs_layout_passes=False,
            skip_device_barrier=True,
        ),
    )(tok16, table_t)
    return out[:EMB]


# speedup vs baseline: 32.1560x; 1.0471x over previous
import jax
import jax.numpy as jnp
from jax import lax
from jax.experimental import pallas as pl
from jax.experimental.pallas import tpu as pltpu
from jax.experimental.pallas import tpu_sc as plsc

EMB = 64
LANES = 16
BLK = 128


def _sc_lookup(tok_hbm, table_hbm, out_hbm, tok_v, tile_v, out_v, sem):
    wid = lax.axis_index("s") * 2 + lax.axis_index("c")

    @pl.when(wid == 0)
    def _():
        pltpu.sync_copy(tok_hbm, tok_v)
        tv = tok_v[:]
        lane = lax.rem(tv, jnp.full((LANES,), BLK, jnp.int32))
        blk = lax.div(tv, jnp.full((LANES,), BLK, jnp.int32))[0]
        base = pl.multiple_of(blk * BLK, BLK)
        pltpu.sync_copy(table_hbm.at[:, pl.ds(base, BLK)], tile_v)
        for k in range(EMB // LANES):
            rows = lax.iota(jnp.int32, LANES) + k * LANES
            chunk = plsc.load_gather(tile_v, [rows, lane])
            out_v[pl.ds(k * LANES, LANES)] = chunk
            out_v[pl.ds(EMB + k * LANES, LANES)] = chunk
        pltpu.sync_copy(out_v, out_hbm)


def kernel(table, token):
    table_t = table.T
    tok16 = jnp.broadcast_to(jnp.asarray(token, jnp.int32).reshape(1), (LANES,))
    out = pl.kernel(
        _sc_lookup,
        out_type=jax.ShapeDtypeStruct((2 * EMB,), jnp.float32),
        mesh=plsc.VectorSubcoreMesh(core_axis_name="c", subcore_axis_name="s",
                                    num_cores=1),
        scratch_types=[
            pltpu.VMEM((LANES,), jnp.int32),
            pltpu.VMEM((EMB, BLK), jnp.float32),
            pltpu.VMEM((2 * EMB,), jnp.float32),
            pltpu.SemaphoreType.DMA,
        ],
        compiler_params=pltpu.CompilerParams(
            needs_layout_passes=False,
            skip_device_barrier=True,
        ),
    )(tok16, table_t)
    return out[:EMB]


# speedup vs baseline: 32.5421x; 1.0120x over previous
import jax
import jax.numpy as jnp
from jax import lax
from jax.experimental import pallas as pl
from jax.experimental.pallas import tpu as pltpu
from jax.experimental.pallas import tpu_sc as plsc

EMB = 64
LANES = 16
BLK = 128


def _sc_lookup(tok_hbm, table_hbm, out_hbm, tok_v, tile_v, out_v, sem):
    wid = lax.axis_index("s") * 2 + lax.axis_index("c")

    @pl.when(wid == 0)
    def _():
        pltpu.sync_copy(tok_hbm, tok_v)
        tv = tok_v[:]
        lane = lax.rem(tv, jnp.full((LANES,), BLK, jnp.int32))
        blk = lax.div(tv, jnp.full((LANES,), BLK, jnp.int32))[0]
        base = pl.multiple_of(blk * BLK, BLK)
        pltpu.sync_copy(table_hbm.at[:, pl.ds(base, BLK)], tile_v)
        for k in range(EMB // LANES):
            rows = lax.iota(jnp.int32, LANES) + k * LANES
            chunk = plsc.load_gather(tile_v, [rows, lane])
            out_v[pl.ds(k * LANES, LANES)] = chunk
            out_v[pl.ds(EMB + k * LANES, LANES)] = chunk
        pltpu.sync_copy(out_v, out_hbm)


def kernel(table, token):
    table_t = table.T
    tok16 = jnp.broadcast_to(jnp.asarray(token, jnp.int32).reshape(1), (LANES,))
    out = pl.kernel(
        _sc_lookup,
        out_type=jax.ShapeDtypeStruct((2 * EMB,), jnp.float32),
        mesh=plsc.VectorSubcoreMesh(core_axis_name="c", subcore_axis_name="s",
                                    num_cores=1, num_subcores=1),
        scratch_types=[
            pltpu.VMEM((LANES,), jnp.int32),
            pltpu.VMEM((EMB, BLK), jnp.float32),
            pltpu.VMEM((2 * EMB,), jnp.float32),
            pltpu.SemaphoreType.DMA,
        ],
        compiler_params=pltpu.CompilerParams(
            needs_layout_passes=False,
            skip_device_barrier=True,
        ),
    )(tok16, table_t)
    return out[:EMB]


# speedup vs baseline: 33.3125x; 1.0237x over previous
import jax
import jax.numpy as jnp
from jax import lax
from jax.experimental import pallas as pl
from jax.experimental.pallas import tpu as pltpu
from jax.experimental.pallas import tpu_sc as plsc

EMB = 64
LANES = 16
BLK = 128


def _sc_lookup(tok_hbm, table_hbm, out_hbm, tok_v, tile_v, out_v, sem):
    pltpu.sync_copy(tok_hbm, tok_v)
    tv = tok_v[:]
    lane = lax.bitwise_and(tv, jnp.full((LANES,), BLK - 1, jnp.int32))
    blk = lax.shift_right_logical(tv, jnp.full((LANES,), 7, jnp.int32))[0]
    base = pl.multiple_of(blk * BLK, BLK)
    pltpu.sync_copy(table_hbm.at[:, pl.ds(base, BLK)], tile_v)
    for k in range(EMB // LANES):
        rows = lax.iota(jnp.int32, LANES) + k * LANES
        chunk = plsc.load_gather(tile_v, [rows, lane])
        out_v[pl.ds(k * LANES, LANES)] = chunk
        out_v[pl.ds(EMB + k * LANES, LANES)] = chunk
    pltpu.sync_copy(out_v, out_hbm)


def kernel(table, token):
    table_t = table.T
    tok16 = jnp.broadcast_to(jnp.asarray(token, jnp.int32).reshape(1), (LANES,))
    out = pl.kernel(
        _sc_lookup,
        out_type=jax.ShapeDtypeStruct((2 * EMB,), jnp.float32),
        mesh=plsc.VectorSubcoreMesh(core_axis_name="c", subcore_axis_name="s",
                                    num_cores=1, num_subcores=1),
        scratch_types=[
            pltpu.VMEM((LANES,), jnp.int32),
            pltpu.VMEM((EMB, BLK), jnp.float32),
            pltpu.VMEM((2 * EMB,), jnp.float32),
            pltpu.SemaphoreType.DMA,
        ],
        compiler_params=pltpu.CompilerParams(
            needs_layout_passes=False,
            skip_device_barrier=True,
            disable_bounds_checks=True,
            disable_semaphore_checks=True,
        ),
    )(tok16, table_t)
    return out[:EMB]
